# Initial kernel scaffold; baseline (speedup 1.0000x reference)
import jax
import jax.numpy as jnp
from jax.experimental import pallas as pl


def kernel(embeddings, targets):
    raise NotImplementedError("write your pallas kernel here")


# speedup vs baseline: 2.1346x; 2.1346x over previous
import functools

import jax
import jax.numpy as jnp
from jax.experimental import pallas as pl
from jax.experimental.pallas import tpu as pltpu

MARGIN_ = 1.0
B_ = 4096
RB_ = 512
NB_ = B_ // RB_


def _triplet_block_kernel(emb_row_ref, emb_all_ref, tgt_row_ref, tgt_col_ref,
                          out_ref, acc_ref):
    i = pl.program_id(0)

    @pl.when(i == 0)
    def _init():
        acc_ref[0] = 0.0
        acc_ref[1] = 0.0
        acc_ref[2] = 0.0

    e_row = emb_row_ref[...]
    e_all = emb_all_ref[...]
    sq_row = jnp.sum(e_row * e_row, axis=1, keepdims=True)
    sq_all = jnp.sum(e_all * e_all, axis=1, keepdims=True).T
    gram = jax.lax.dot_general(
        e_row, e_all,
        dimension_numbers=(((1,), (1,)), ((), ())),
        preferred_element_type=jnp.float32,
    )
    S = jnp.maximum(sq_row + sq_all - 2.0 * gram, 0.0)

    t_all = tgt_row_ref[...]
    t_row = tgt_col_ref[...]
    same = t_row == t_all

    s_neg = jnp.where(same, jnp.inf, S)
    s_an = jnp.min(s_neg, axis=1, keepdims=True)

    t = jnp.sqrt(s_an) - MARGIN_
    cond = (t < 0.0) | (S > t * t)

    col = jax.lax.broadcasted_iota(jnp.int32, (RB_, B_), 1)
    row = jax.lax.broadcasted_iota(jnp.int32, (RB_, B_), 0) + i * RB_
    tri = same & (col > row) & cond

    losses = jnp.maximum(S - s_an + MARGIN_, 0.0)
    zero = jnp.zeros_like(S)
    loss_part = jnp.sum(jnp.where(tri, losses, zero))
    cnt_part = jnp.sum(jnp.where(tri, jnp.ones_like(S), zero))
    acc_part = jnp.sum(jnp.where(tri & (S < s_an), jnp.ones_like(S), zero))

    acc_ref[0] += loss_part
    acc_ref[1] += cnt_part
    acc_ref[2] += acc_part

    @pl.when(i == NB_ - 1)
    def _finish():
        cnt = acc_ref[1]
        out_ref[0] = acc_ref[0] / cnt
        out_ref[1] = acc_ref[2] / cnt


@functools.partial(jax.jit, static_argnames=())
def _run(embeddings, targets):
    tgt_row = targets.astype(jnp.int32).reshape(1, B_)
    tgt_col = targets.astype(jnp.int32).reshape(B_, 1)
    out = pl.pallas_call(
        _triplet_block_kernel,
        grid=(NB_,),
        in_specs=[
            pl.BlockSpec((RB_, 128), lambda i: (i, 0)),
            pl.BlockSpec((B_, 128), lambda i: (0, 0)),
            pl.BlockSpec((1, B_), lambda i: (0, 0)),
            pl.BlockSpec((RB_, 1), lambda i: (i, 0)),
        ],
        out_specs=pl.BlockSpec(memory_space=pltpu.SMEM),
        out_shape=jax.ShapeDtypeStruct((2,), jnp.float32),
        scratch_shapes=[pltpu.SMEM((3,), jnp.float32)],
    )(embeddings, embeddings, tgt_row, tgt_col)
    return out[0], out[1]


def kernel(embeddings, targets):
    loss, accuracy = _run(embeddings, targets)
    return loss.reshape(()), accuracy.reshape(())
